# baseline (speedup 1.0000x reference)
import functools

import jax
import jax.numpy as jnp
from jax import lax
from jax.experimental import pallas as pl
from jax.experimental.pallas import tpu as pltpu
from jax.experimental.pallas import tpu_sc as plsc

D_MODEL = 768
N_ROWS = 4 * 8192

_info = plsc.get_sparse_core_info()
_NC, _NS = _info.num_cores, _info.num_subcores
_NW = _NC * _NS
_ROWS_PER_W = N_ROWS // _NW
_CHUNK = 16
_NCHUNK = _ROWS_PER_W // _CHUNK
_NBUF = 4


def _make_sc_call():
    mesh = plsc.VectorSubcoreMesh(core_axis_name="c", subcore_axis_name="s")

    @functools.partial(
        pl.kernel,
        out_type=jax.ShapeDtypeStruct((N_ROWS, D_MODEL), jnp.float32),
        mesh=mesh,
        compiler_params=pltpu.CompilerParams(needs_layout_passes=False),
        scratch_types=[
            pltpu.VMEM((_ROWS_PER_W,), jnp.int32),
            pltpu.VMEM((_NBUF, _CHUNK, D_MODEL), jnp.float32),
            pltpu.VMEM((_NBUF, _CHUNK, D_MODEL // 2), jnp.int32),
            pltpu.SemaphoreType.DMA((_NBUF,)),
            pltpu.SemaphoreType.DMA((_NBUF,)),
            pltpu.SemaphoreType.DMA((_NBUF,)),
        ],
    )
    def sc_add_pe(
        x_hbm, idx_hbm, pe_hbm, out_hbm, idx_all, acc_v, pe_v, sem_x, sem_pe, sem_out
    ):
        wid = lax.axis_index("s") * _NC + lax.axis_index("c")
        base = wid * _ROWS_PER_W

        w_per_row = 8192 // _ROWS_PER_W
        pltpu.sync_copy(
            idx_hbm.at[wid // w_per_row, pl.ds((wid % w_per_row) * _ROWS_PER_W, _ROWS_PER_W)],
            idx_all,
        )

        def issue_in(g, b):
            off = base + g * _CHUNK
            pltpu.async_copy(x_hbm.at[pl.ds(off, _CHUNK)], acc_v.at[b], sem_x.at[b])
            pltpu.async_copy(
                pe_hbm.at[idx_all.at[pl.ds(g * _CHUNK, _CHUNK)]],
                pe_v.at[b],
                sem_pe.at[b],
            )

        def wait_in(g, b):
            off = base + g * _CHUNK
            pltpu.make_async_copy(
                x_hbm.at[pl.ds(off, _CHUNK)], acc_v.at[b], sem_x.at[b]
            ).wait()
            pltpu.make_async_copy(
                pe_hbm.at[idx_all.at[pl.ds(g * _CHUNK, _CHUNK)]],
                pe_v.at[b],
                sem_pe.at[b],
            ).wait()

        def issue_out(g, b):
            off = base + g * _CHUNK
            pltpu.async_copy(acc_v.at[b], out_hbm.at[pl.ds(off, _CHUNK)], sem_out.at[b])

        def wait_out(g, b):
            off = base + g * _CHUNK
            pltpu.make_async_copy(
                acc_v.at[b], out_hbm.at[pl.ds(off, _CHUNK)], sem_out.at[b]
            ).wait()

        issue_in(0, 0)
        issue_in(1, 1)

        def outer(gg, carry):
            for j in range(_NBUF):
                g = gg * _NBUF + j
                wait_in(g, j)

                b2 = (j + 2) % _NBUF
                if j >= 2:
                    wait_out(g - 2, b2)

                    @pl.when(gg < _NCHUNK // _NBUF - 1)
                    def _():
                        issue_in(g + 2, b2)

                else:

                    @pl.when(gg > 0)
                    def _():
                        wait_out(g - 2, b2)

                    issue_in(g + 2, b2)

                @plsc.parallel_loop(0, _CHUNK)
                def row_body(r, j=j):
                    for k in range(D_MODEL // 32):
                        w = plsc.bitcast(
                            pe_v[j, r, pl.ds(k * 16, 16)], jnp.bfloat16
                        )
                        a, b = plsc.unpack(
                            w,
                            format=plsc.PackFormat.INTERLEAVED,
                            preferred_element_type=jnp.float32,
                        )
                        plsc.addupdate(acc_v.at[j, r, pl.ds(k * 32, 16)], a)
                        plsc.addupdate(acc_v.at[j, r, pl.ds(k * 32 + 16, 16)], b)
                issue_out(g, j)
            return carry

        lax.fori_loop(0, _NCHUNK // _NBUF, outer, 0)
        wait_out(_NCHUNK - 2, (_NCHUNK - 2) % _NBUF)
        wait_out(_NCHUNK - 1, (_NCHUNK - 1) % _NBUF)

    return sc_add_pe


_sc_add_pe = _make_sc_call()


def kernel(x, position_ids, pe):
    b, s, d = x.shape
    xf = x.reshape(b * s, d)
    ids = position_ids.astype(jnp.int32)
    m = pe.shape[0]
    pe_bf = pe.astype(jnp.bfloat16).reshape(m, d // 32, 2, 16)
    pe_i32 = lax.bitcast_convert_type(pe_bf.transpose(0, 1, 3, 2), jnp.int32)
    pe_i32 = pe_i32.reshape(m, d // 2)
    out = _sc_add_pe(xf, ids, pe_i32)
    return out.reshape(b, s, d)


# speedup vs baseline: 1.2837x; 1.2837x over previous
import functools

import jax
import jax.numpy as jnp
from jax import lax
from jax.experimental import pallas as pl
from jax.experimental.pallas import tpu as pltpu
from jax.experimental.pallas import tpu_sc as plsc

D_MODEL = 768
N_ROWS = 4 * 8192

_info = plsc.get_sparse_core_info()
_NC, _NS = _info.num_cores, _info.num_subcores
_NW = _NC * _NS
_ROWS_PER_W = N_ROWS // _NW
_CHUNK = 8
_NCHUNK = _ROWS_PER_W // _CHUNK
_NBUF = 8
_DIST = 4


def _make_sc_call():
    mesh = plsc.VectorSubcoreMesh(core_axis_name="c", subcore_axis_name="s")

    @functools.partial(
        pl.kernel,
        out_type=jax.ShapeDtypeStruct((N_ROWS, D_MODEL), jnp.float32),
        mesh=mesh,
        scratch_types=[
            pltpu.VMEM((_ROWS_PER_W,), jnp.int32),
            pltpu.VMEM((_NBUF, _CHUNK, D_MODEL), jnp.float32),
            pltpu.VMEM((_NBUF, _CHUNK, D_MODEL), jnp.float32),
            pltpu.SemaphoreType.DMA((_NBUF,)),
            pltpu.SemaphoreType.DMA((_NBUF,)),
            pltpu.SemaphoreType.DMA((_NBUF,)),
        ],
    )
    def sc_add_pe(
        x_hbm, idx_hbm, pe_hbm, out_hbm, idx_all, acc_v, pe_v, sem_x, sem_pe, sem_out
    ):
        wid = lax.axis_index("s") * _NC + lax.axis_index("c")
        base = wid * _ROWS_PER_W

        w_per_row = 8192 // _ROWS_PER_W
        pltpu.sync_copy(
            idx_hbm.at[wid // w_per_row, pl.ds((wid % w_per_row) * _ROWS_PER_W, _ROWS_PER_W)],
            idx_all,
        )

        def issue_in(g, b):
            off = base + g * _CHUNK
            pltpu.async_copy(x_hbm.at[pl.ds(off, _CHUNK)], acc_v.at[b], sem_x.at[b])
            pltpu.async_copy(
                pe_hbm.at[idx_all.at[pl.ds(g * _CHUNK, _CHUNK)]],
                pe_v.at[b],
                sem_pe.at[b],
            )

        def wait_in(g, b):
            off = base + g * _CHUNK
            pltpu.make_async_copy(
                x_hbm.at[pl.ds(off, _CHUNK)], acc_v.at[b], sem_x.at[b]
            ).wait()
            pltpu.make_async_copy(
                pe_hbm.at[idx_all.at[pl.ds(g * _CHUNK, _CHUNK)]],
                pe_v.at[b],
                sem_pe.at[b],
            ).wait()

        def issue_out(g, b):
            off = base + g * _CHUNK
            pltpu.async_copy(acc_v.at[b], out_hbm.at[pl.ds(off, _CHUNK)], sem_out.at[b])

        def wait_out(g, b):
            off = base + g * _CHUNK
            pltpu.make_async_copy(
                acc_v.at[b], out_hbm.at[pl.ds(off, _CHUNK)], sem_out.at[b]
            ).wait()

        for d in range(_DIST):
            issue_in(d, d)

        def outer(gg, carry):
            for j in range(_NBUF):
                g = gg * _NBUF + j
                wait_in(g, j)

                b2 = (j + _DIST) % _NBUF
                if j >= _NBUF - _DIST:
                    wait_out(g + _DIST - _NBUF, b2)

                    @pl.when(gg < _NCHUNK // _NBUF - 1)
                    def _():
                        issue_in(g + _DIST, b2)

                else:

                    @pl.when(gg > 0)
                    def _():
                        wait_out(g + _DIST - _NBUF, b2)

                    issue_in(g + _DIST, b2)

                @plsc.parallel_loop(0, _CHUNK)
                def row_body(r, j=j):
                    for i in range(D_MODEL // 16):
                        sl = pl.ds(i * 16, 16)
                        plsc.addupdate(acc_v.at[j, r, sl], pe_v[j, r, sl])
                issue_out(g, j)
            return carry

        lax.fori_loop(0, _NCHUNK // _NBUF, outer, 0)
        for c in range(_NCHUNK - (_NBUF - _DIST), _NCHUNK):
            wait_out(c, c % _NBUF)

    return sc_add_pe


_sc_add_pe = _make_sc_call()


def kernel(x, position_ids, pe):
    b, s, d = x.shape
    xf = x.reshape(b * s, d)
    ids = position_ids.astype(jnp.int32)
    out = _sc_add_pe(xf, ids, pe)
    return out.reshape(b, s, d)


# speedup vs baseline: 1.2853x; 1.0012x over previous
import functools

import jax
import jax.numpy as jnp
from jax import lax
from jax.experimental import pallas as pl
from jax.experimental.pallas import tpu as pltpu
from jax.experimental.pallas import tpu_sc as plsc

D_MODEL = 768
N_ROWS = 4 * 8192

_info = plsc.get_sparse_core_info()
_NC, _NS = _info.num_cores, _info.num_subcores
_NW = _NC * _NS
_ROWS_PER_W = N_ROWS // _NW
_CHUNK = 8
_NCHUNK = _ROWS_PER_W // _CHUNK
_NBUF = 8
_DIST = 4


def _make_sc_call():
    mesh = plsc.VectorSubcoreMesh(core_axis_name="c", subcore_axis_name="s")

    @functools.partial(
        pl.kernel,
        out_type=jax.ShapeDtypeStruct((N_ROWS, D_MODEL), jnp.float32),
        mesh=mesh,
        scratch_types=[
            pltpu.VMEM((_ROWS_PER_W,), jnp.int32),
            pltpu.VMEM((_NBUF, _CHUNK, D_MODEL), jnp.float32),
            pltpu.VMEM((_NBUF, _CHUNK, D_MODEL), jnp.float32),
            pltpu.SemaphoreType.DMA((_NBUF,)),
            pltpu.SemaphoreType.DMA((_NBUF,)),
            pltpu.SemaphoreType.DMA((_NBUF,)),
        ],
    )
    def sc_add_pe(
        x_hbm, idx_hbm, pe_hbm, out_hbm, idx_all, acc_v, pe_v, sem_x, sem_pe, sem_out
    ):
        wid = lax.axis_index("s") * _NC + lax.axis_index("c")
        base = wid * _ROWS_PER_W

        w_per_row = 8192 // _ROWS_PER_W
        pltpu.sync_copy(
            idx_hbm.at[wid // w_per_row, pl.ds((wid % w_per_row) * _ROWS_PER_W, _ROWS_PER_W)],
            idx_all,
        )

        def issue_in(g, b):
            off = base + g * _CHUNK
            pltpu.async_copy(x_hbm.at[pl.ds(off, _CHUNK)], acc_v.at[b], sem_x.at[b])
            pltpu.async_copy(
                pe_hbm.at[idx_all.at[pl.ds(g * _CHUNK, _CHUNK)]],
                pe_v.at[b],
                sem_pe.at[b],
            )

        def wait_in(g, b):
            off = base + g * _CHUNK
            pltpu.make_async_copy(
                x_hbm.at[pl.ds(off, _CHUNK)], acc_v.at[b], sem_x.at[b]
            ).wait()
            pltpu.make_async_copy(
                pe_hbm.at[idx_all.at[pl.ds(g * _CHUNK, _CHUNK)]],
                pe_v.at[b],
                sem_pe.at[b],
            ).wait()

        def issue_out(g, b):
            off = base + g * _CHUNK
            pltpu.async_copy(acc_v.at[b], out_hbm.at[pl.ds(off, _CHUNK)], sem_out.at[b])

        def wait_out(g, b):
            off = base + g * _CHUNK
            pltpu.make_async_copy(
                acc_v.at[b], out_hbm.at[pl.ds(off, _CHUNK)], sem_out.at[b]
            ).wait()

        for d in range(_DIST):
            issue_in(d, d)

        def outer(gg, carry):
            for j in range(_NBUF):
                g = gg * _NBUF + j
                wait_in(g, j)

                b2 = (j + _DIST) % _NBUF
                if j >= _NBUF - _DIST:
                    wait_out(g + _DIST - _NBUF, b2)

                    @pl.when(gg < _NCHUNK // _NBUF - 1)
                    def _():
                        issue_in(g + _DIST, b2)

                else:

                    @pl.when(gg > 0)
                    def _():
                        wait_out(g + _DIST - _NBUF, b2)

                    issue_in(g + _DIST, b2)

                def row_body(r, c, j=j):
                    for i in range(D_MODEL // 16):
                        sl = pl.ds(i * 16, 16)
                        plsc.addupdate(acc_v.at[j, r, sl], pe_v[j, r, sl])
                    return c

                lax.fori_loop(0, _CHUNK, row_body, 0)
                issue_out(g, j)
            return carry

        lax.fori_loop(0, _NCHUNK // _NBUF, outer, 0)
        for c in range(_NCHUNK - (_NBUF - _DIST), _NCHUNK):
            wait_out(c, c % _NBUF)

    return sc_add_pe


_sc_add_pe = _make_sc_call()


def kernel(x, position_ids, pe):
    b, s, d = x.shape
    xf = x.reshape(b * s, d)
    ids = position_ids.astype(jnp.int32)
    out = _sc_add_pe(xf, ids, pe)
    return out.reshape(b, s, d)
